# Initial kernel scaffold; baseline (speedup 1.0000x reference)
import jax
import jax.numpy as jnp
from jax.experimental import pallas as pl


def kernel(x, e, global_attr, node_graph_ids, edge_graph_ids, W, b):
    raise NotImplementedError("write your pallas kernel here")


# speedup vs baseline: 5.2516x; 5.2516x over previous
import functools

import jax
import jax.numpy as jnp
from jax import lax
from jax.experimental import pallas as pl
from jax.experimental.pallas import tpu as pltpu
from jax.experimental.pallas import tpu_sc as plsc

NUM_GRAPHS = 64
E_ROWS = 320000
E_FEATS = 16
N_ROWS = 10000
X_FEATS = 128
BATCH = 128
E_BATCHES = E_ROWS // BATCH
N_MAIN = (N_ROWS // BATCH) * BATCH
N_BATCHES = N_MAIN // BATCH
N_TAIL = N_ROWS - N_MAIN
NC, NS = 2, 16
NW = NC * NS
E_PER_W = E_BATCHES // NW
E_CHUNK = 13
N_PER_W = 2


def _sc_segment_sums(e, ids_e, x_main, ids_n, z_e, z_x):
    mesh = plsc.VectorSubcoreMesh(
        core_axis_name="c", subcore_axis_name="s", num_cores=NC, num_subcores=NS
    )

    @functools.partial(
        pl.kernel,
        out_type=(
            jax.ShapeDtypeStruct((NC, NUM_GRAPHS, E_FEATS), jnp.float32),
            jax.ShapeDtypeStruct((NC, NUM_GRAPHS, X_FEATS), jnp.float32),
        ),
        mesh=mesh,
        compiler_params=pltpu.CompilerParams(use_tc_tiling_on_sc=False),
        scratch_types=[
            pltpu.VMEM((E_CHUNK * BATCH, E_FEATS), jnp.float32),
            pltpu.VMEM((E_CHUNK, BATCH), jnp.int32),
            pltpu.VMEM((BATCH, X_FEATS), jnp.float32),
            pltpu.VMEM((1, BATCH), jnp.int32),
            pltpu.VMEM_SHARED((NUM_GRAPHS, E_FEATS), jnp.float32),
            pltpu.VMEM_SHARED((NUM_GRAPHS, X_FEATS), jnp.float32),
        ],
    )
    def seg_kernel(e_hbm, ids_e_hbm, x_hbm, ids_n_hbm, ze_hbm, zx_hbm,
                   out_e, out_x, ebuf, eids, xbuf, xids, esum, xsum):
        cid = lax.axis_index("c")
        sid = lax.axis_index("s")
        wid = sid * NC + cid

        @pl.when(sid == 0)
        def _init():
            pltpu.sync_copy(ze_hbm, esum)
            pltpu.sync_copy(zx_hbm, xsum)

        plsc.subcore_barrier()

        ebase = wid * E_PER_W + jnp.maximum(wid - (NW - 4), 0)

        for ch in range(E_PER_W // E_CHUNK):
            cb = ebase + ch * E_CHUNK
            pltpu.sync_copy(e_hbm.at[pl.ds(cb * BATCH, E_CHUNK * BATCH), :], ebuf)
            pltpu.sync_copy(ids_e_hbm.at[pl.ds(cb, E_CHUNK), :], eids)
            for j in range(E_CHUNK):
                pltpu.sync_copy(ebuf.at[pl.ds(j * BATCH, BATCH), :],
                                esum.at[eids.at[j]], add=True)

        @pl.when(wid >= NW - 4)
        def _edge_extra():
            eb = ebase + E_PER_W
            pltpu.sync_copy(e_hbm.at[pl.ds(eb * BATCH, BATCH), :],
                            ebuf.at[pl.ds(0, BATCH), :])
            pltpu.sync_copy(ids_e_hbm.at[pl.ds(eb, 1), :], eids.at[pl.ds(0, 1), :])
            pltpu.sync_copy(ebuf.at[pl.ds(0, BATCH), :],
                            esum.at[eids.at[0]], add=True)

        for k in range(N_PER_W):
            nb = wid * N_PER_W + k
            pltpu.sync_copy(x_hbm.at[pl.ds(nb * BATCH, BATCH), :], xbuf)
            pltpu.sync_copy(ids_n_hbm.at[pl.ds(nb, 1), :], xids)
            pltpu.sync_copy(xbuf, xsum.at[xids.at[0]], add=True)

        @pl.when(wid < N_BATCHES - N_PER_W * NW)
        def _node_extra():
            nb = N_PER_W * NW + wid
            pltpu.sync_copy(x_hbm.at[pl.ds(nb * BATCH, BATCH), :], xbuf)
            pltpu.sync_copy(ids_n_hbm.at[pl.ds(nb, 1), :], xids)
            pltpu.sync_copy(xbuf, xsum.at[xids.at[0]], add=True)

        plsc.subcore_barrier()

        @pl.when(sid == 0)
        def _publish():
            pltpu.sync_copy(esum, out_e.at[cid])
            pltpu.sync_copy(xsum, out_x.at[cid])

    return seg_kernel(e, ids_e, x_main, ids_n, z_e, z_x)


def _tc_finish(ep, xp, x_tail, tail_ids, g, W, b2):
    def body(ep_ref, xp_ref, xt_ref, idt_ref, g_ref, w_ref, b_ref, o_ref):
        he = ep_ref[0] + ep_ref[1]
        hx = xp_ref[0] + xp_ref[1]
        ids = idt_ref[...]
        oh = (ids == lax.broadcasted_iota(jnp.int32, (N_TAIL, NUM_GRAPHS), 1)
              ).astype(jnp.float32)
        hx = hx + lax.dot_general(
            oh, xt_ref[...], (((0,), (0,)), ((), ())),
            preferred_element_type=jnp.float32, precision=lax.Precision.HIGHEST)
        w1 = w_ref[0:E_FEATS, :]
        w2 = w_ref[E_FEATS:E_FEATS + X_FEATS, :]
        w3 = w_ref[E_FEATS + X_FEATS:, :]
        acc = jnp.dot(he, w1, preferred_element_type=jnp.float32,
                      precision=lax.Precision.HIGHEST)
        acc = acc + jnp.dot(hx, w2, preferred_element_type=jnp.float32,
                            precision=lax.Precision.HIGHEST)
        acc = acc + jnp.dot(g_ref[...], w3, preferred_element_type=jnp.float32,
                            precision=lax.Precision.HIGHEST)
        o_ref[...] = acc + b_ref[...]

    return pl.pallas_call(
        body,
        out_shape=jax.ShapeDtypeStruct((NUM_GRAPHS, X_FEATS), jnp.float32),
    )(ep, xp, x_tail, tail_ids, g, W, b2)


def kernel(x, e, global_attr, node_graph_ids, edge_graph_ids, W, b):
    ids_e = edge_graph_ids.astype(jnp.int32).reshape(E_BATCHES, BATCH)
    ids_n = node_graph_ids[:N_MAIN].astype(jnp.int32).reshape(N_BATCHES, BATCH)
    tail_ids = node_graph_ids[N_MAIN:].astype(jnp.int32).reshape(N_TAIL, 1)
    z_e = jnp.zeros((NUM_GRAPHS, E_FEATS), jnp.float32)
    z_x = jnp.zeros((NUM_GRAPHS, X_FEATS), jnp.float32)
    ep, xp = _sc_segment_sums(e, ids_e, x, ids_n, z_e, z_x)
    return _tc_finish(ep, xp, x[N_MAIN:], tail_ids, global_attr, W,
                      b.reshape(1, X_FEATS))


# speedup vs baseline: 5.4878x; 1.0450x over previous
import functools

import jax
import jax.numpy as jnp
from jax import lax
from jax.experimental import pallas as pl
from jax.experimental.pallas import tpu as pltpu
from jax.experimental.pallas import tpu_sc as plsc

NUM_GRAPHS = 64
TRASH = NUM_GRAPHS
ACC_ROWS = 72
E_ROWS = 320000
E_FEATS = 16
PACK = 8
P_ROWS = E_ROWS // PACK
N_ROWS = 10000
X_FEATS = 128
BATCH = 128
NC, NS = 2, 16
NW = NC * NS
P_PER_W = 1280
P_LAST = P_ROWS - (NW - 1) * P_PER_W
G_PER_W = P_PER_W // 16
G_LAST = P_LAST // 16
CHUNK = 256
N_BATCHES = N_ROWS // BATCH
N_MAIN = N_BATCHES * BATCH
N_TAIL = N_ROWS - N_MAIN
N_PER_W = 2


def _sc_segment_sums(e2, ids_e, x_main, ids_n, z_e, z_e16, z_x):
    mesh = plsc.VectorSubcoreMesh(
        core_axis_name="c", subcore_axis_name="s", num_cores=NC, num_subcores=NS
    )

    @functools.partial(
        pl.kernel,
        out_type=(
            jax.ShapeDtypeStruct((NC, ACC_ROWS, X_FEATS), jnp.float32),
            jax.ShapeDtypeStruct((NC, ACC_ROWS, E_FEATS), jnp.float32),
            jax.ShapeDtypeStruct((NC, NUM_GRAPHS, X_FEATS), jnp.float32),
        ),
        mesh=mesh,
        compiler_params=pltpu.CompilerParams(use_tc_tiling_on_sc=False,
                                             needs_layout_passes=False),
        scratch_types=[
            pltpu.VMEM((CHUNK, X_FEATS), jnp.float32),
            pltpu.VMEM((G_PER_W, BATCH), jnp.int32),
            pltpu.VMEM((P_PER_W // BATCH, BATCH), jnp.int32),
            pltpu.VMEM((16, E_FEATS), jnp.float32),
            pltpu.VMEM((1, 16), jnp.int32),
            pltpu.VMEM((BATCH, X_FEATS), jnp.float32),
            pltpu.VMEM((N_BATCHES + 2, BATCH), jnp.int32),
            pltpu.VMEM_SHARED((ACC_ROWS, X_FEATS), jnp.float32),
            pltpu.VMEM_SHARED((ACC_ROWS, E_FEATS), jnp.float32),
            pltpu.VMEM_SHARED((NUM_GRAPHS, X_FEATS), jnp.float32),
        ],
    )
    def seg_kernel(e_hbm, ids_e_hbm, x_hbm, ids_n_hbm, ze_hbm, ze16_hbm,
                   zx_hbm, out_e, out_e16, out_x, ebuf, eids, pidx, stg, sidx,
                   xbuf, xids, esum, esum16, xsum):
        cid = lax.axis_index("c")
        sid = lax.axis_index("s")
        wid = sid * NC + cid

        @pl.when(sid == 0)
        def _init():
            pltpu.sync_copy(ze_hbm, esum)
            pltpu.sync_copy(ze16_hbm, esum16)
            pltpu.sync_copy(zx_hbm, xsum)

        plsc.subcore_barrier()

        @pl.when(wid < NW - 1)
        def _ids_full():
            pltpu.sync_copy(ids_e_hbm.at[pl.ds(wid * G_PER_W, G_PER_W), :], eids)

        @pl.when(wid == NW - 1)
        def _ids_last():
            pltpu.sync_copy(ids_e_hbm.at[pl.ds((NW - 1) * G_PER_W, G_LAST), :],
                            eids.at[pl.ds(0, G_LAST), :])

        pltpu.sync_copy(ids_n_hbm, xids)

        ng = jnp.where(wid == NW - 1, G_LAST, G_PER_W)
        iota16 = lax.broadcasted_iota(jnp.int32, (16,), 0)
        offs_h = iota16 * PACK
        offs_t = offs_h + (PACK - 1)
        trash_v = jnp.full((16,), TRASH, jnp.int32)

        def _purity(g, _):
            heads = plsc.load_gather(eids.at[g], [offs_h])
            tails = plsc.load_gather(eids.at[g], [offs_t])
            valid = g < ng
            pure = jnp.logical_and(heads == tails, valid)
            idxv = jnp.where(pure, heads, trash_v)
            pidx[g // PACK, pl.ds((g % PACK) * 16, 16)] = idxv
            return _

        lax.fori_loop(0, G_PER_W, _purity, None)

        def _fix_pack(gv, kv, p_locv):
            col = jnp.minimum(PACK * kv + iota16, BATCH - 1)
            gath = plsc.load_gather(eids, [gv, col])
            idxv = jnp.where(iota16 < PACK, gath, trash_v)
            sidx[0, :] = idxv
            for m in range(PACK):
                stg[m, :] = plsc.load_gather(ebuf, [p_locv, m * E_FEATS + iota16])
            pltpu.sync_copy(stg, esum16.at[sidx.at[0]], add=True)

        def _impure_scan(glo, ghi, chunk_pack_base):
            def _g_body(g, _):
                pvv = pidx[g // PACK, pl.ds((g % PACK) * 16, 16)]
                mask0 = pvv == TRASH

                def _cond(mask):
                    return jnp.any(mask)

                def _mask_body(mask):
                    kv = plsc.all_reduce_ffs(mask)
                    gv = jnp.full((16,), 0, jnp.int32) + g
                    _fix_pack(gv, kv, 16 * g + kv - chunk_pack_base)
                    return jnp.logical_and(mask, iota16 != kv)

                lax.while_loop(_cond, _mask_body, mask0)
                return _
            lax.fori_loop(glo, ghi, _g_body, None)

        pbase = wid * P_PER_W
        for c in range(P_PER_W // CHUNK):
            def _chunk(c=c):
                cb = pbase + c * CHUNK
                pltpu.sync_copy(e_hbm.at[pl.ds(cb, CHUNK), :], ebuf)
                for j in range(CHUNK // BATCH):
                    pltpu.sync_copy(ebuf.at[pl.ds(j * BATCH, BATCH), :],
                                    esum.at[pidx.at[c * (CHUNK // BATCH) + j]],
                                    add=True)
                _impure_scan(c * (CHUNK // 16), (c + 1) * (CHUNK // 16),
                             c * CHUNK)
            if c == 0:
                _chunk()
            else:
                pl.when(wid < NW - 1)(_chunk)

        @pl.when(wid == NW - 1)
        def _edge_last():
            cb = (NW - 1) * P_PER_W + CHUNK
            rem = P_LAST - CHUNK
            pltpu.sync_copy(e_hbm.at[pl.ds(cb, rem), :], ebuf.at[pl.ds(0, rem), :])
            pltpu.sync_copy(ebuf.at[pl.ds(0, BATCH), :],
                            esum.at[pidx.at[CHUNK // BATCH]], add=True)
            _impure_scan(CHUNK // 16, G_LAST, CHUNK)

        for k in range(N_PER_W):
            nb = wid * N_PER_W + k
            pltpu.sync_copy(x_hbm.at[pl.ds(nb * BATCH, BATCH), :], xbuf)
            pltpu.sync_copy(xbuf, xsum.at[xids.at[nb]], add=True)

        @pl.when(wid < N_BATCHES - N_PER_W * NW)
        def _node_extra():
            nb = N_PER_W * NW + wid
            pltpu.sync_copy(x_hbm.at[pl.ds(nb * BATCH, BATCH), :], xbuf)
            pltpu.sync_copy(xbuf, xsum.at[xids.at[nb]], add=True)

        plsc.subcore_barrier()

        @pl.when(sid == 0)
        def _publish():
            pltpu.sync_copy(esum, out_e.at[cid])
            pltpu.sync_copy(esum16, out_e16.at[cid])
            pltpu.sync_copy(xsum, out_x.at[cid])

    return seg_kernel(e2, ids_e, x_main, ids_n, z_e, z_e16, z_x)


def _tc_finish(ep, ep16, xp, x_tail, tail_nids, g, W, b2):
    def body(ep_ref, ep16_ref, xp_ref, xt_ref, idt_ref, g_ref, w_ref, b_ref,
             o_ref):
        es = ep_ref[0, 0:NUM_GRAPHS, :] + ep_ref[1, 0:NUM_GRAPHS, :]
        he = es[:, 0:E_FEATS]
        he = he + ep16_ref[0, 0:NUM_GRAPHS, :] + ep16_ref[1, 0:NUM_GRAPHS, :]
        for m in range(1, PACK):
            he = he + es[:, m * E_FEATS:(m + 1) * E_FEATS]
        hx = xp_ref[0] + xp_ref[1]
        ohn = (idt_ref[...] == lax.broadcasted_iota(
            jnp.int32, (N_TAIL, NUM_GRAPHS), 1)).astype(jnp.float32)
        hx = hx + lax.dot_general(
            ohn, xt_ref[...], (((0,), (0,)), ((), ())),
            preferred_element_type=jnp.float32, precision=lax.Precision.HIGHEST)
        w1 = w_ref[0:E_FEATS, :]
        w2 = w_ref[E_FEATS:E_FEATS + X_FEATS, :]
        w3 = w_ref[E_FEATS + X_FEATS:, :]
        acc = jnp.dot(he, w1, preferred_element_type=jnp.float32,
                      precision=lax.Precision.HIGHEST)
        acc = acc + jnp.dot(hx, w2, preferred_element_type=jnp.float32,
                            precision=lax.Precision.HIGHEST)
        acc = acc + jnp.dot(g_ref[...], w3, preferred_element_type=jnp.float32,
                            precision=lax.Precision.HIGHEST)
        o_ref[...] = acc + b_ref[...]

    return pl.pallas_call(
        body,
        out_shape=jax.ShapeDtypeStruct((NUM_GRAPHS, X_FEATS), jnp.float32),
    )(ep, ep16, xp, x_tail, tail_nids, g, W, b2)


def kernel(x, e, global_attr, node_graph_ids, edge_graph_ids, W, b):
    e2 = e.reshape(P_ROWS, PACK * E_FEATS)
    ids_e = edge_graph_ids.astype(jnp.int32).reshape(E_ROWS // BATCH, BATCH)
    ids_n = node_graph_ids[:N_MAIN].astype(jnp.int32).reshape(N_BATCHES, BATCH)
    ids_n = jnp.concatenate([ids_n, jnp.zeros((2, BATCH), jnp.int32)], 0)
    tail_nids = node_graph_ids[N_MAIN:].astype(jnp.int32).reshape(N_TAIL, 1)
    z_e = jnp.zeros((ACC_ROWS, X_FEATS), jnp.float32)
    z_e16 = jnp.zeros((ACC_ROWS, E_FEATS), jnp.float32)
    z_x = jnp.zeros((NUM_GRAPHS, X_FEATS), jnp.float32)
    ep, ep16, xp = _sc_segment_sums(e2, ids_e, x, ids_n, z_e, z_e16, z_x)
    return _tc_finish(ep, ep16, xp, x[N_MAIN:], tail_nids, global_attr, W,
                      b.reshape(1, X_FEATS))


# speedup vs baseline: 5.5032x; 1.0028x over previous
import functools

import jax
import jax.numpy as jnp
from jax import lax
from jax.experimental import pallas as pl
from jax.experimental.pallas import tpu as pltpu
from jax.experimental.pallas import tpu_sc as plsc

NUM_GRAPHS = 64
TRASH = NUM_GRAPHS
E_ACC = 72
ACC_ROWS = E_ACC + NUM_GRAPHS
E_ROWS = 320000
E_FEATS = 16
PACK = 8
P_ROWS = E_ROWS // PACK
N_ROWS = 10000
X_FEATS = 128
BATCH = 128
NC, NS = 2, 16
NW = NC * NS
P_PER_W = 1280
P_LAST = P_ROWS - (NW - 1) * P_PER_W
G_PER_W = P_PER_W // 16
G_LAST = P_LAST // 16
CHUNK = 256
E_IDROWS = E_ROWS // BATCH
N_BATCHES = N_ROWS // BATCH
N_MAIN = N_BATCHES * BATCH
N_TAIL = N_ROWS - N_MAIN
N_PER_W = 2


def _sc_segment_sums(e2, ids_all, x_main, z_ex, z_16):
    mesh = plsc.VectorSubcoreMesh(
        core_axis_name="c", subcore_axis_name="s", num_cores=NC, num_subcores=NS
    )

    @functools.partial(
        pl.kernel,
        out_type=(
            jax.ShapeDtypeStruct((NC, ACC_ROWS, X_FEATS), jnp.float32),
            jax.ShapeDtypeStruct((NC, E_ACC, E_FEATS), jnp.float32),
        ),
        mesh=mesh,
        compiler_params=pltpu.CompilerParams(use_tc_tiling_on_sc=False,
                                             needs_layout_passes=False),
        scratch_types=[
            pltpu.VMEM((CHUNK, X_FEATS), jnp.float32),
            pltpu.VMEM((G_PER_W, BATCH), jnp.int32),
            pltpu.VMEM((P_PER_W // BATCH, BATCH), jnp.int32),
            pltpu.VMEM((16, X_FEATS), jnp.float32),
            pltpu.VMEM((16, E_FEATS), jnp.float32),
            pltpu.VMEM((1, 16), jnp.int32),
            pltpu.VMEM((1, 16), jnp.int32),
            pltpu.VMEM((BATCH, X_FEATS), jnp.float32),
            pltpu.VMEM((N_BATCHES + 2, BATCH), jnp.int32),
            pltpu.VMEM_SHARED((ACC_ROWS, X_FEATS), jnp.float32),
            pltpu.VMEM_SHARED((E_ACC, E_FEATS), jnp.float32),
        ],
    )
    def seg_kernel(e_hbm, ids_hbm, x_hbm, zex_hbm, z16_hbm,
                   out_ex, out_16, ebuf, eids, pidx, gbuf, stg, sidx, srow,
                   xbuf, xids, acc, acc16):
        cid = lax.axis_index("c")
        sid = lax.axis_index("s")
        wid = sid * NC + cid

        @pl.when(sid == 0)
        def _init():
            pltpu.sync_copy(zex_hbm, acc)
            pltpu.sync_copy(z16_hbm, acc16)

        plsc.subcore_barrier()

        @pl.when(wid < NW - 1)
        def _ids_full():
            pltpu.sync_copy(ids_hbm.at[pl.ds(wid * G_PER_W, G_PER_W), :], eids)

        @pl.when(wid == NW - 1)
        def _ids_last():
            pltpu.sync_copy(ids_hbm.at[pl.ds((NW - 1) * G_PER_W, G_LAST), :],
                            eids.at[pl.ds(0, G_LAST), :])

        pltpu.sync_copy(ids_hbm.at[pl.ds(E_IDROWS, N_BATCHES + 2), :], xids)

        ng = jnp.where(wid == NW - 1, G_LAST, G_PER_W)
        iota16 = lax.broadcasted_iota(jnp.int32, (16,), 0)
        trash_v = jnp.full((16,), TRASH, jnp.int32)
        pbase = wid * P_PER_W

        def _purity(g, _):
            heads = plsc.load_gather(eids.at[g], [iota16 * PACK])
            tails = plsc.load_gather(eids.at[g], [iota16 * PACK + (PACK - 1)])
            pure = jnp.logical_and(heads == tails, g < ng)
            pidx[g // PACK, pl.ds((g % PACK) * 16, 16)] = jnp.where(
                pure, heads, trash_v)
            return _

        lax.fori_loop(0, G_PER_W, _purity, None)

        def _chunk(c, _):
            pltpu.sync_copy(e_hbm.at[pl.ds(pbase + c * CHUNK, CHUNK), :], ebuf)
            for j in range(CHUNK // BATCH):
                pltpu.sync_copy(ebuf.at[pl.ds(j * BATCH, BATCH), :],
                                acc.at[pidx.at[c * (CHUNK // BATCH) + j]],
                                add=True)
            return _

        lax.fori_loop(0, jnp.where(wid == NW - 1, 1, P_PER_W // CHUNK),
                      _chunk, None)

        @pl.when(wid == NW - 1)
        def _edge_last():
            rem = P_LAST - CHUNK
            pltpu.sync_copy(e_hbm.at[pl.ds(pbase + CHUNK, rem), :],
                            ebuf.at[pl.ds(0, rem), :])
            pltpu.sync_copy(ebuf.at[pl.ds(0, BATCH), :],
                            acc.at[pidx.at[CHUNK // BATCH]], add=True)

        def _impure(g, _):
            pvv = pidx[g // PACK, pl.ds((g % PACK) * 16, 16)]
            mask0 = pvv == TRASH

            @pl.when(jnp.any(jnp.logical_and(mask0, g < ng)))
            def _():
                srow[0, :] = pbase + 16 * g + iota16
                pltpu.sync_copy(e_hbm.at[srow.at[0]], gbuf)

                def _fix(mask):
                    kv = plsc.all_reduce_ffs(mask)
                    col = jnp.minimum(PACK * kv + iota16, BATCH - 1)
                    gath = plsc.load_gather(eids, [iota16 * 0 + g, col])
                    sidx[0, :] = jnp.where(iota16 < PACK, gath, trash_v)
                    for m in range(PACK):
                        stg[m, :] = plsc.load_gather(
                            gbuf, [kv, m * E_FEATS + iota16])
                    pltpu.sync_copy(stg, acc16.at[sidx.at[0]], add=True)
                    return jnp.logical_and(mask, iota16 != kv)

                lax.while_loop(lambda m: jnp.any(m), _fix, mask0)
            return _

        lax.fori_loop(0, ng, _impure, None)

        def _node(nb):
            pltpu.sync_copy(x_hbm.at[pl.ds(nb * BATCH, BATCH), :], xbuf)
            pltpu.sync_copy(xbuf, acc.at[xids.at[nb]], add=True)

        def _node_k(k, _):
            _node(wid * N_PER_W + k)
            return _

        lax.fori_loop(0, N_PER_W, _node_k, None)

        @pl.when(wid < N_BATCHES - N_PER_W * NW)
        def _node_extra():
            _node(N_PER_W * NW + wid)

        plsc.subcore_barrier()

        @pl.when(sid == 0)
        def _publish():
            pltpu.sync_copy(acc, out_ex.at[cid])
            pltpu.sync_copy(acc16, out_16.at[cid])

    return seg_kernel(e2, ids_all, x_main, z_ex, z_16)


def _tc_finish(ep, ep16, x_tail, tail_nids, g, W, b2):
    def body(ep_ref, ep16_ref, xt_ref, idt_ref, g_ref, w_ref, b_ref, o_ref):
        es = ep_ref[0, 0:NUM_GRAPHS, :] + ep_ref[1, 0:NUM_GRAPHS, :]
        he = ep16_ref[0, 0:NUM_GRAPHS, :] + ep16_ref[1, 0:NUM_GRAPHS, :]
        for m in range(PACK):
            he = he + es[:, m * E_FEATS:(m + 1) * E_FEATS]
        hx = (ep_ref[0, E_ACC:ACC_ROWS, :] + ep_ref[1, E_ACC:ACC_ROWS, :])
        ohn = (idt_ref[...] == lax.broadcasted_iota(
            jnp.int32, (N_TAIL, NUM_GRAPHS), 1)).astype(jnp.float32)
        hx = hx + lax.dot_general(
            ohn, xt_ref[...], (((0,), (0,)), ((), ())),
            preferred_element_type=jnp.float32, precision=lax.Precision.HIGHEST)
        w1 = w_ref[0:E_FEATS, :]
        w2 = w_ref[E_FEATS:E_FEATS + X_FEATS, :]
        w3 = w_ref[E_FEATS + X_FEATS:, :]
        acc = jnp.dot(he, w1, preferred_element_type=jnp.float32,
                      precision=lax.Precision.HIGHEST)
        acc = acc + jnp.dot(hx, w2, preferred_element_type=jnp.float32,
                            precision=lax.Precision.HIGHEST)
        acc = acc + jnp.dot(g_ref[...], w3, preferred_element_type=jnp.float32,
                            precision=lax.Precision.HIGHEST)
        o_ref[...] = acc + b_ref[...]

    return pl.pallas_call(
        body,
        out_shape=jax.ShapeDtypeStruct((NUM_GRAPHS, X_FEATS), jnp.float32),
    )(ep, ep16, x_tail, tail_nids, g, W, b2)


def kernel(x, e, global_attr, node_graph_ids, edge_graph_ids, W, b):
    e2 = e.reshape(P_ROWS, PACK * E_FEATS)
    ids_e = edge_graph_ids.astype(jnp.int32).reshape(E_IDROWS, BATCH)
    ids_n = node_graph_ids[:N_MAIN].astype(jnp.int32).reshape(N_BATCHES, BATCH)
    ids_all = jnp.concatenate(
        [ids_e, ids_n + E_ACC, jnp.zeros((2, BATCH), jnp.int32)], 0)
    tail_nids = node_graph_ids[N_MAIN:].astype(jnp.int32).reshape(N_TAIL, 1)
    z_ex = jnp.zeros((ACC_ROWS, X_FEATS), jnp.float32)
    z_16 = jnp.zeros((E_ACC, E_FEATS), jnp.float32)
    ep, ep16 = _sc_segment_sums(e2, ids_all, x, z_ex, z_16)
    return _tc_finish(ep, ep16, x[N_MAIN:], tail_nids, global_attr, W,
                      b.reshape(1, X_FEATS))


# speedup vs baseline: 7.9277x; 1.4406x over previous
import functools

import jax
import jax.numpy as jnp
from jax import lax
from jax.experimental import pallas as pl
from jax.experimental.pallas import tpu as pltpu
from jax.experimental.pallas import tpu_sc as plsc

NUM_GRAPHS = 64
E_ROWS = 320000
E_FEATS = 16
N_ROWS = 10000
X_FEATS = 128
BATCH = 128
NC, NS = 2, 16
NW = NC * NS
N_BATCHES = N_ROWS // BATCH
N_MAIN = N_BATCHES * BATCH
N_TAIL = N_ROWS - N_MAIN
N_PER_W = 2
E_CHUNK = 2560
E_STEPS = E_ROWS // E_CHUNK
E_IDROWS = E_CHUNK // BATCH


def _sc_node_sums(x_main, ids_n, z_x):
    mesh = plsc.VectorSubcoreMesh(
        core_axis_name="c", subcore_axis_name="s", num_cores=NC, num_subcores=NS
    )

    @functools.partial(
        pl.kernel,
        out_type=jax.ShapeDtypeStruct((NC, NUM_GRAPHS, X_FEATS), jnp.float32),
        mesh=mesh,
        compiler_params=pltpu.CompilerParams(use_tc_tiling_on_sc=False,
                                             needs_layout_passes=False),
        scratch_types=[
            pltpu.VMEM((BATCH, X_FEATS), jnp.float32),
            pltpu.VMEM((N_BATCHES + 2, BATCH), jnp.int32),
            pltpu.VMEM_SHARED((NUM_GRAPHS, X_FEATS), jnp.float32),
        ],
    )
    def node_kernel(x_hbm, ids_hbm, zx_hbm, out_x, xbuf, xids, acc):
        cid = lax.axis_index("c")
        sid = lax.axis_index("s")
        wid = sid * NC + cid

        @pl.when(sid == 0)
        def _init():
            pltpu.sync_copy(zx_hbm, acc)

        plsc.subcore_barrier()

        pltpu.sync_copy(ids_hbm, xids)

        def _node(nb):
            pltpu.sync_copy(x_hbm.at[pl.ds(nb * BATCH, BATCH), :], xbuf)
            pltpu.sync_copy(xbuf, acc.at[xids.at[nb]], add=True)

        def _node_k(k, _):
            _node(wid * N_PER_W + k)
            return _

        lax.fori_loop(0, N_PER_W, _node_k, None)

        @pl.when(wid < N_BATCHES - N_PER_W * NW)
        def _node_extra():
            _node(N_PER_W * NW + wid)

        plsc.subcore_barrier()

        @pl.when(sid == 0)
        def _publish():
            pltpu.sync_copy(acc, out_x.at[cid])

    return node_kernel(x_main, ids_n, z_x)


def _tc_edge_sums(eT, ids3d):
    def body(ids_ref, et_ref, he_ref, ohg, acc):
        i = pl.program_id(0)

        @pl.when(i == 0)
        def _init():
            acc[...] = jnp.zeros((NUM_GRAPHS, E_FEATS), jnp.float32)

        idb = ids_ref[0]
        giota = lax.broadcasted_iota(jnp.int32, (NUM_GRAPHS, 1), 0)
        for c in range(E_IDROWS):
            row = idb[c:c + 1, :]
            ohg[:, c * BATCH:(c + 1) * BATCH] = (
                row == giota).astype(jnp.float32)
        acc[...] = acc[...] + lax.dot_general(
            ohg[...], et_ref[...], (((1,), (1,)), ((), ())),
            preferred_element_type=jnp.float32, precision=lax.Precision.HIGHEST)

        @pl.when(i == E_STEPS - 1)
        def _out():
            he_ref[...] = acc[...]

    return pl.pallas_call(
        body,
        grid=(E_STEPS,),
        in_specs=[
            pl.BlockSpec((1, E_IDROWS, BATCH), lambda i: (i, 0, 0)),
            pl.BlockSpec((E_FEATS, E_CHUNK), lambda i: (0, i)),
        ],
        out_specs=pl.BlockSpec((NUM_GRAPHS, E_FEATS), lambda i: (0, 0)),
        out_shape=jax.ShapeDtypeStruct((NUM_GRAPHS, E_FEATS), jnp.float32),
        scratch_shapes=[
            pltpu.VMEM((NUM_GRAPHS, E_CHUNK), jnp.float32),
            pltpu.VMEM((NUM_GRAPHS, E_FEATS), jnp.float32),
        ],
        compiler_params=pltpu.CompilerParams(
            dimension_semantics=("arbitrary",)),
    )(ids3d, eT)


def _tc_finish(he, xp, x_tail, tail_nids, g, W, b2):
    def body(he_ref, xp_ref, xt_ref, idt_ref, g_ref, w_ref, b_ref, o_ref):
        he = he_ref[...]
        hx = xp_ref[0] + xp_ref[1]
        ohn = (idt_ref[...] == lax.broadcasted_iota(
            jnp.int32, (N_TAIL, NUM_GRAPHS), 1)).astype(jnp.float32)
        hx = hx + lax.dot_general(
            ohn, xt_ref[...], (((0,), (0,)), ((), ())),
            preferred_element_type=jnp.float32, precision=lax.Precision.HIGHEST)
        w1 = w_ref[0:E_FEATS, :]
        w2 = w_ref[E_FEATS:E_FEATS + X_FEATS, :]
        w3 = w_ref[E_FEATS + X_FEATS:, :]
        acc = jnp.dot(he, w1, preferred_element_type=jnp.float32,
                      precision=lax.Precision.HIGHEST)
        acc = acc + jnp.dot(hx, w2, preferred_element_type=jnp.float32,
                            precision=lax.Precision.HIGHEST)
        acc = acc + jnp.dot(g_ref[...], w3, preferred_element_type=jnp.float32,
                            precision=lax.Precision.HIGHEST)
        o_ref[...] = acc + b_ref[...]

    return pl.pallas_call(
        body,
        out_shape=jax.ShapeDtypeStruct((NUM_GRAPHS, X_FEATS), jnp.float32),
    )(he, xp, x_tail, tail_nids, g, W, b2)


def kernel(x, e, global_attr, node_graph_ids, edge_graph_ids, W, b):
    eT = e.T
    ids3d = edge_graph_ids.astype(jnp.int32).reshape(E_STEPS, E_IDROWS, BATCH)
    ids_n = node_graph_ids[:N_MAIN].astype(jnp.int32).reshape(N_BATCHES, BATCH)
    ids_n = jnp.concatenate([ids_n, jnp.zeros((2, BATCH), jnp.int32)], 0)
    tail_nids = node_graph_ids[N_MAIN:].astype(jnp.int32).reshape(N_TAIL, 1)
    z_x = jnp.zeros((NUM_GRAPHS, X_FEATS), jnp.float32)
    xp = _sc_node_sums(x, ids_n, z_x)
    he = _tc_edge_sums(eT, ids3d)
    return _tc_finish(he, xp, x[N_MAIN:], tail_nids, global_attr, W,
                      b.reshape(1, X_FEATS))
